# baseline (speedup 1.0000x reference)
import functools

import jax
import jax.numpy as jnp
from jax import lax
from jax.experimental import pallas as pl
from jax.experimental.pallas import tpu as pltpu
from jax.experimental.pallas import tpu_sc as plsc

N = 10000
E = 320000
D = 128
EPS = 1e-5

NC = 2
NS = 16
NW = NC * NS
EDGES_RAW = E // NW
CHUNK = 96
NCHUNK = 105
EDGES_PER_TILE = NCHUNK * CHUNK
PASS = 32
NFULL = NCHUNK // PASS
RING = 3
NP = 10240
ROWS_PER_TILE = NP // NS
LANES = 16
DSTEP = D // LANES
_ROWBLOCKS = [(t * CHUNK, CHUNK) for t in range(ROWS_PER_TILE // CHUNK)]
_ROWBLOCKS.append((ROWS_PER_TILE // CHUNK * CHUNK,
                   ROWS_PER_TILE - ROWS_PER_TILE // CHUNK * CHUNK))


def _spmm_body(x_hbm, src_hbm, dst_hbm, val_hbm, out_hbm,
               src_v, dst_v, val_v, rows0_v, rows1_v, rows2_v,
               acc_sh, gsem0, gsem1, gsem2, ssem0, ssem1, ssem2):
    c = lax.axis_index("c")
    s = lax.axis_index("s")
    wid = c * NS + s
    rows = (rows0_v, rows1_v, rows2_v)
    gsem = (gsem0, gsem1, gsem2)
    ssem = (ssem0, ssem1, ssem2)

    def _start_gather(k, b):
        pltpu.async_copy(x_hbm.at[src_v.at[pl.ds(k * CHUNK, CHUNK)]],
                         rows[b], gsem[b])

    def _wait_gather(k, b):
        pltpu.make_async_copy(x_hbm.at[src_v.at[pl.ds(k * CHUNK, CHUNK)]],
                              rows[b], gsem[b]).wait()

    def _start_scatter(k, b):
        pltpu.async_copy(rows[b], acc_sh.at[dst_v.at[k]], ssem[b], add=True)

    def _wait_scatter(k, b):
        pltpu.make_async_copy(rows[b], acc_sh.at[dst_v.at[k]], ssem[b]).wait()

    def _scale(k, b):
        rb = rows[b]
        def _group(g, carry):
            vv = val_v[pl.ds(k * CHUNK + g * LANES, LANES)]
            for e in range(LANES):
                v = vv[e]
                i = g * LANES + e
                for j in range(DSTEP):
                    sl = pl.ds(j * LANES, LANES)
                    rb[i, sl] = rb[i, sl] * v
            return carry
        lax.fori_loop(0, CHUNK // LANES, _group, 0)

    def _zrow(r, carry):
        for j in range(DSTEP):
            rows0_v[r, pl.ds(j * LANES, LANES)] = jnp.zeros((LANES,),
                                                            jnp.float32)
        return carry
    lax.fori_loop(0, CHUNK, _zrow, 0)
    for base, nrow in _ROWBLOCKS:
        pltpu.sync_copy(rows0_v.at[pl.ds(0, nrow)],
                        acc_sh.at[pl.ds(s * ROWS_PER_TILE + base, nrow)])
    plsc.subcore_barrier()

    ebase = wid * EDGES_PER_TILE

    def _run_pass(koff, npass):
        pltpu.sync_copy(src_hbm.at[pl.ds(ebase + koff * CHUNK, npass * CHUNK)],
                        src_v.at[pl.ds(0, npass * CHUNK)])
        pltpu.sync_copy(val_hbm.at[pl.ds(ebase + koff * CHUNK, npass * CHUNK)],
                        val_v.at[pl.ds(0, npass * CHUNK)])
        pltpu.sync_copy(dst_hbm.at[wid, pl.ds(koff, npass)],
                        dst_v.at[pl.ds(0, npass)])

        _start_gather(0, 0)
        ntrip = npass // RING

        def _triple(t, carry):
            for b in range(RING):
                k = RING * t + b
                nb = (b + 1) % RING
                if b < 2:
                    @pl.when(t >= 1)
                    def _():
                        _wait_scatter(k - 2, nb)
                    _start_gather(k + 1, nb)
                else:
                    @pl.when(RING * t + RING < npass)
                    def _():
                        _wait_scatter(k - 2, nb)
                        _start_gather(k + 1, nb)
                _wait_gather(k, b)
                _scale(k, b)
                _start_scatter(k, b)
            return carry
        lax.fori_loop(0, ntrip, _triple, 0)

        for k in range(RING * ntrip, npass):
            b = k % RING
            if k + 1 < npass:
                _wait_scatter(k - 2, (b + 1) % RING)
                _start_gather(k + 1, (b + 1) % RING)
            _wait_gather(k, b)
            _scale(k, b)
            _start_scatter(k, b)
        for k in range(npass - RING, npass):
            _wait_scatter(k, k % RING)

    def _pass_loop(p, carry):
        _run_pass(pl.multiple_of(p * PASS, PASS), PASS)
        return carry
    lax.fori_loop(0, NFULL, _pass_loop, 0)
    _run_pass(NFULL * PASS, NCHUNK - NFULL * PASS)

    plsc.subcore_barrier()
    for base, nrow in _ROWBLOCKS:
        b0 = s * ROWS_PER_TILE + base
        pltpu.sync_copy(acc_sh.at[pl.ds(b0, nrow)], rows0_v.at[pl.ds(0, nrow)])
        pltpu.sync_copy(rows0_v.at[pl.ds(0, nrow)],
                        out_hbm.at[c, pl.ds(b0, nrow)])


_spmm_call = functools.partial(
    pl.kernel,
    out_type=jax.ShapeDtypeStruct((NC, NP, D), jnp.float32),
    mesh=plsc.VectorSubcoreMesh(core_axis_name="c", subcore_axis_name="s"),
    scratch_types=[
        pltpu.VMEM((PASS * CHUNK,), jnp.int32),
        pltpu.VMEM((PASS, CHUNK), jnp.int32),
        pltpu.VMEM((PASS * CHUNK,), jnp.float32),
        pltpu.VMEM((CHUNK, D), jnp.float32),
        pltpu.VMEM((CHUNK, D), jnp.float32),
        pltpu.VMEM((CHUNK, D), jnp.float32),
        pltpu.VMEM_SHARED((NP, D), jnp.float32),
        pltpu.SemaphoreType.DMA,
        pltpu.SemaphoreType.DMA,
        pltpu.SemaphoreType.DMA,
        pltpu.SemaphoreType.DMA,
        pltpu.SemaphoreType.DMA,
        pltpu.SemaphoreType.DMA,
    ],
)(_spmm_body)


def _spmm(x, src1d, dst3d, val1d):
    return _spmm_call(x, src1d, dst3d, val1d)


def _bn_body(x_ref, g_ref, b_ref, o_ref):
    x = x_ref[...]
    mu = jnp.mean(x, axis=0, keepdims=True)
    xc = x - mu
    var = jnp.mean(xc * xc, axis=0, keepdims=True)
    o_ref[...] = g_ref[...] * (xc * lax.rsqrt(var + EPS)) + b_ref[...]


def _combine_body(p_ref, o_ref):
    o_ref[...] = p_ref[0, :N, :] + p_ref[1, :N, :]


def _mid_body(xn_ref, y1_ref, y2p_ref, w_ref, b_ref, g_ref, be_ref, o_ref):
    wa = w_ref[0] - w_ref[2]
    wb = w_ref[1]
    wc = 2.0 * w_ref[2]
    o = (jnp.dot(xn_ref[...], wa, preferred_element_type=jnp.float32)
         + jnp.dot(y1_ref[...], wb, preferred_element_type=jnp.float32)
         + jnp.dot(y2p_ref[0, :N, :] + y2p_ref[1, :N, :], wc,
                   preferred_element_type=jnp.float32)
         + b_ref[...])
    o = jnp.maximum(o, 0.0)
    mu = jnp.mean(o, axis=0, keepdims=True)
    oc = o - mu
    var = jnp.mean(oc * oc, axis=0, keepdims=True)
    o_ref[...] = g_ref[...] * (oc * lax.rsqrt(var + EPS)) + be_ref[...]


def _final_body(z_ref, y3_ref, y4p_ref, w_ref, b_ref, xn_ref, o_ref):
    wa = w_ref[0] - w_ref[2]
    wb = w_ref[1]
    wc = 2.0 * w_ref[2]
    o = (jnp.dot(z_ref[...], wa, preferred_element_type=jnp.float32)
         + jnp.dot(y3_ref[...], wb, preferred_element_type=jnp.float32)
         + jnp.dot(y4p_ref[0, :N, :] + y4p_ref[1, :N, :], wc,
                   preferred_element_type=jnp.float32)
         + b_ref[...])
    o_ref[...] = jnp.maximum(o + xn_ref[...], 0.0)


_ND = jax.ShapeDtypeStruct((N, D), jnp.float32)


def kernel(x, lap_indices, lap_values, g1, be1, W1, b1, g2, be2, W2, b2):
    pad = ((0, 0), (0, EDGES_PER_TILE - EDGES_RAW))
    src1d = jnp.pad(lap_indices[1].reshape(NW, EDGES_RAW), pad).reshape(-1)
    dst3d = jnp.pad(lap_indices[0].reshape(NW, EDGES_RAW),
                    pad).reshape(NW, NCHUNK, CHUNK)
    val1d = jnp.pad(lap_values.reshape(NW, EDGES_RAW), pad).reshape(-1)
    g1r, be1r, b1r = g1.reshape(1, D), be1.reshape(1, D), b1.reshape(1, D)
    g2r, be2r, b2r = g2.reshape(1, D), be2.reshape(1, D), b2.reshape(1, D)

    xn = pl.pallas_call(_bn_body, out_shape=_ND)(x, g1r, be1r)
    y1p = _spmm(xn, src1d, dst3d, val1d)
    y1 = pl.pallas_call(_combine_body, out_shape=_ND)(y1p)
    y2p = _spmm(y1, src1d, dst3d, val1d)
    z = pl.pallas_call(_mid_body, out_shape=_ND)(
        xn, y1, y2p, W1, b1r, g2r, be2r)
    y3p = _spmm(z, src1d, dst3d, val1d)
    y3 = pl.pallas_call(_combine_body, out_shape=_ND)(y3p)
    y4p = _spmm(y3, src1d, dst3d, val1d)
    out = pl.pallas_call(_final_body, out_shape=_ND)(
        z, y3, y4p, W2, b2r, xn)
    return out


# speedup vs baseline: 1.6319x; 1.6319x over previous
import functools

import jax
import jax.numpy as jnp
from jax import lax
from jax.experimental import pallas as pl
from jax.experimental.pallas import tpu as pltpu
from jax.experimental.pallas import tpu_sc as plsc

N = 10000
E = 320000
D = 128
EPS = 1e-5

NC = 2
NS = 16
NW = NC * NS
EDGES_RAW = E // NW
CHUNK = 80
NCHUNK = 125
EDGES_PER_TILE = NCHUNK * CHUNK
PASS = 48
NFULL = NCHUNK // PASS
RING = 3
NP = 10240
ROWS_PER_TILE = NP // NS
LANES = 16
DSTEP = D // LANES
_ROWBLOCKS = [(t * CHUNK, CHUNK) for t in range(ROWS_PER_TILE // CHUNK)]
if ROWS_PER_TILE % CHUNK:
    _ROWBLOCKS.append((ROWS_PER_TILE // CHUNK * CHUNK,
                       ROWS_PER_TILE % CHUNK))


def _spmm_body(x_hbm, src_hbm, dst_hbm, val_hbm, out_hbm,
               src_v, dst_v, val_v, rows0_v, rows1_v, rows2_v,
               acc_sh, gsem0, gsem1, gsem2, ssem0, ssem1, ssem2):
    c = lax.axis_index("c")
    s = lax.axis_index("s")
    wid = c * NS + s
    rows = (rows0_v, rows1_v, rows2_v)
    gsem = (gsem0, gsem1, gsem2)
    ssem = (ssem0, ssem1, ssem2)

    def _start_gather(k, b):
        pltpu.async_copy(x_hbm.at[src_v.at[pl.ds(k * CHUNK, CHUNK)]],
                         rows[b], gsem[b])

    def _wait_gather(k, b):
        pltpu.make_async_copy(x_hbm.at[src_v.at[pl.ds(k * CHUNK, CHUNK)]],
                              rows[b], gsem[b]).wait()

    def _start_scatter(k, b):
        pltpu.async_copy(rows[b], acc_sh.at[dst_v.at[k]], ssem[b], add=True)

    def _wait_scatter(k, b):
        pltpu.make_async_copy(rows[b], acc_sh.at[dst_v.at[k]], ssem[b]).wait()

    def _scale(k, b):
        rb = rows[b]
        def _group(g, carry):
            vv = val_v[pl.ds(k * CHUNK + g * LANES, LANES)]
            for e in range(LANES):
                v = vv[e]
                i = g * LANES + e
                for j in range(DSTEP):
                    sl = pl.ds(j * LANES, LANES)
                    rb[i, sl] = rb[i, sl] * v
            return carry
        lax.fori_loop(0, CHUNK // LANES, _group, 0)

    def _zrow(r, carry):
        for j in range(DSTEP):
            rows0_v[r, pl.ds(j * LANES, LANES)] = jnp.zeros((LANES,),
                                                            jnp.float32)
        return carry
    lax.fori_loop(0, CHUNK, _zrow, 0)
    for base, nrow in _ROWBLOCKS:
        pltpu.sync_copy(rows0_v.at[pl.ds(0, nrow)],
                        acc_sh.at[pl.ds(s * ROWS_PER_TILE + base, nrow)])
    plsc.subcore_barrier()

    ebase = wid * EDGES_PER_TILE

    def _run_pass(koff, npass):
        pltpu.sync_copy(src_hbm.at[pl.ds(ebase + koff * CHUNK, npass * CHUNK)],
                        src_v.at[pl.ds(0, npass * CHUNK)])
        pltpu.sync_copy(val_hbm.at[pl.ds(ebase + koff * CHUNK, npass * CHUNK)],
                        val_v.at[pl.ds(0, npass * CHUNK)])
        pltpu.sync_copy(dst_hbm.at[wid, pl.ds(koff, npass)],
                        dst_v.at[pl.ds(0, npass)])

        _start_gather(0, 0)
        ntrip = npass // RING

        def _triple(t, carry):
            for b in range(RING):
                k = RING * t + b
                nb = (b + 1) % RING
                if b < 2:
                    @pl.when(t >= 1)
                    def _():
                        _wait_scatter(k - 2, nb)
                    _start_gather(k + 1, nb)
                else:
                    @pl.when(RING * t + RING < npass)
                    def _():
                        _wait_scatter(k - 2, nb)
                        _start_gather(k + 1, nb)
                _wait_gather(k, b)
                _scale(k, b)
                _start_scatter(k, b)
            return carry
        lax.fori_loop(0, ntrip, _triple, 0)

        for k in range(RING * ntrip, npass):
            b = k % RING
            if k + 1 < npass:
                _wait_scatter(k - 2, (b + 1) % RING)
                _start_gather(k + 1, (b + 1) % RING)
            _wait_gather(k, b)
            _scale(k, b)
            _start_scatter(k, b)
        for k in range(npass - RING, npass):
            _wait_scatter(k, k % RING)

    def _pass_loop(p, carry):
        _run_pass(pl.multiple_of(p * PASS, PASS), PASS)
        return carry
    lax.fori_loop(0, NFULL, _pass_loop, 0)
    _run_pass(NFULL * PASS, NCHUNK - NFULL * PASS)

    plsc.subcore_barrier()
    for base, nrow in _ROWBLOCKS:
        b0 = s * ROWS_PER_TILE + base
        pltpu.sync_copy(acc_sh.at[pl.ds(b0, nrow)], rows0_v.at[pl.ds(0, nrow)])
        pltpu.sync_copy(rows0_v.at[pl.ds(0, nrow)],
                        out_hbm.at[c, pl.ds(b0, nrow)])


_spmm_call = functools.partial(
    pl.kernel,
    out_type=jax.ShapeDtypeStruct((NC, NP, D), jnp.float32),
    mesh=plsc.VectorSubcoreMesh(core_axis_name="c", subcore_axis_name="s"),
    scratch_types=[
        pltpu.VMEM((PASS * CHUNK,), jnp.int32),
        pltpu.VMEM((PASS, CHUNK), jnp.int32),
        pltpu.VMEM((PASS * CHUNK,), jnp.float32),
        pltpu.VMEM((CHUNK, D), jnp.float32),
        pltpu.VMEM((CHUNK, D), jnp.float32),
        pltpu.VMEM((CHUNK, D), jnp.float32),
        pltpu.VMEM_SHARED((NP, D), jnp.float32),
        pltpu.SemaphoreType.DMA,
        pltpu.SemaphoreType.DMA,
        pltpu.SemaphoreType.DMA,
        pltpu.SemaphoreType.DMA,
        pltpu.SemaphoreType.DMA,
        pltpu.SemaphoreType.DMA,
    ],
)(_spmm_body)


def _spmm(x, src1d, dst3d, val1d):
    return _spmm_call(x, src1d, dst3d, val1d)


def _bn_body(x_ref, g_ref, b_ref, o_ref):
    x = x_ref[...]
    mu = jnp.mean(x, axis=0, keepdims=True)
    xc = x - mu
    var = jnp.mean(xc * xc, axis=0, keepdims=True)
    o_ref[...] = g_ref[...] * (xc * lax.rsqrt(var + EPS)) + b_ref[...]


def _combine_body(p_ref, o_ref):
    o_ref[...] = p_ref[0, :N, :] + p_ref[1, :N, :]


def _mid_body(xn_ref, y1_ref, y2p_ref, w_ref, b_ref, g_ref, be_ref, o_ref):
    wa = w_ref[0] - w_ref[2]
    wb = w_ref[1]
    wc = 2.0 * w_ref[2]
    o = (jnp.dot(xn_ref[...], wa, preferred_element_type=jnp.float32)
         + jnp.dot(y1_ref[...], wb, preferred_element_type=jnp.float32)
         + jnp.dot(y2p_ref[0, :N, :] + y2p_ref[1, :N, :], wc,
                   preferred_element_type=jnp.float32)
         + b_ref[...])
    o = jnp.maximum(o, 0.0)
    mu = jnp.mean(o, axis=0, keepdims=True)
    oc = o - mu
    var = jnp.mean(oc * oc, axis=0, keepdims=True)
    o_ref[...] = g_ref[...] * (oc * lax.rsqrt(var + EPS)) + be_ref[...]


def _final_body(z_ref, y3_ref, y4p_ref, w_ref, b_ref, xn_ref, o_ref):
    wa = w_ref[0] - w_ref[2]
    wb = w_ref[1]
    wc = 2.0 * w_ref[2]
    o = (jnp.dot(z_ref[...], wa, preferred_element_type=jnp.float32)
         + jnp.dot(y3_ref[...], wb, preferred_element_type=jnp.float32)
         + jnp.dot(y4p_ref[0, :N, :] + y4p_ref[1, :N, :], wc,
                   preferred_element_type=jnp.float32)
         + b_ref[...])
    o_ref[...] = jnp.maximum(o + xn_ref[...], 0.0)


_ND = jax.ShapeDtypeStruct((N, D), jnp.float32)


def kernel(x, lap_indices, lap_values, g1, be1, W1, b1, g2, be2, W2, b2):
    src1d = lap_indices[1]
    dst3d = lap_indices[0].reshape(NW, NCHUNK, CHUNK)
    val1d = lap_values
    g1r, be1r, b1r = g1.reshape(1, D), be1.reshape(1, D), b1.reshape(1, D)
    g2r, be2r, b2r = g2.reshape(1, D), be2.reshape(1, D), b2.reshape(1, D)

    xn = pl.pallas_call(_bn_body, out_shape=_ND)(x, g1r, be1r)
    y1p = _spmm(xn, src1d, dst3d, val1d)
    y1 = pl.pallas_call(_combine_body, out_shape=_ND)(y1p)
    y2p = _spmm(y1, src1d, dst3d, val1d)
    z = pl.pallas_call(_mid_body, out_shape=_ND)(
        xn, y1, y2p, W1, b1r, g2r, be2r)
    y3p = _spmm(z, src1d, dst3d, val1d)
    y3 = pl.pallas_call(_combine_body, out_shape=_ND)(y3p)
    y4p = _spmm(y3, src1d, dst3d, val1d)
    out = pl.pallas_call(_final_body, out_shape=_ND)(
        z, y3, y4p, W2, b2r, xn)
    return out


# speedup vs baseline: 1.6932x; 1.0376x over previous
import functools

import jax
import jax.numpy as jnp
from jax import lax
from jax.experimental import pallas as pl
from jax.experimental.pallas import tpu as pltpu
from jax.experimental.pallas import tpu_sc as plsc

N = 10000
E = 320000
D = 128
EPS = 1e-5

NC = 2
NS = 16
NW = NC * NS
EDGES_RAW = E // NW
CHUNK = 80
NCHUNK = 125
EDGES_PER_TILE = NCHUNK * CHUNK
PASS = 48
NFULL = NCHUNK // PASS
RING = 3
NP = 10240
ROWS_PER_TILE = NP // NS
LANES = 16
DSTEP = D // LANES
_ROWBLOCKS = [(t * CHUNK, CHUNK) for t in range(ROWS_PER_TILE // CHUNK)]
if ROWS_PER_TILE % CHUNK:
    _ROWBLOCKS.append((ROWS_PER_TILE // CHUNK * CHUNK,
                       ROWS_PER_TILE % CHUNK))


def _spmm_body(x_hbm, src_hbm, dst_hbm, val_hbm, out_hbm,
               src_v, dst_v, val_v, rows0_v, rows1_v, rows2_v,
               acc_sh, gsem0, gsem1, gsem2, ssem0, ssem1, ssem2):
    c = lax.axis_index("c")
    s = lax.axis_index("s")
    wid = c * NS + s
    rows = (rows0_v, rows1_v, rows2_v)
    gsem = (gsem0, gsem1, gsem2)
    ssem = (ssem0, ssem1, ssem2)

    def _start_gather(k, b):
        pltpu.async_copy(x_hbm.at[src_v.at[pl.ds(k * CHUNK, CHUNK)]],
                         rows[b], gsem[b])

    def _wait_gather(k, b):
        pltpu.make_async_copy(x_hbm.at[src_v.at[pl.ds(k * CHUNK, CHUNK)]],
                              rows[b], gsem[b]).wait()

    def _start_scatter(k, b):
        pltpu.async_copy(rows[b], acc_sh.at[dst_v.at[k]], ssem[b], add=True)

    def _wait_scatter(k, b):
        pltpu.make_async_copy(rows[b], acc_sh.at[dst_v.at[k]], ssem[b]).wait()

    def _scale(k, b):
        rb = rows[b]
        def _group(g, carry):
            vv = val_v[pl.ds(k * CHUNK + g * LANES, LANES)]
            for e in range(LANES):
                v = vv[e]
                i = g * LANES + e
                for j in range(DSTEP):
                    sl = pl.ds(j * LANES, LANES)
                    rb[i, sl] = rb[i, sl] * v
            return carry
        lax.fori_loop(0, CHUNK // LANES, _group, 0)

    def _zrow(r, carry):
        for j in range(DSTEP):
            rows0_v[r, pl.ds(j * LANES, LANES)] = jnp.zeros((LANES,),
                                                            jnp.float32)
        return carry
    lax.fori_loop(0, CHUNK, _zrow, 0)
    for base, nrow in _ROWBLOCKS:
        pltpu.async_copy(rows0_v.at[pl.ds(0, nrow)],
                         acc_sh.at[pl.ds(s * ROWS_PER_TILE + base, nrow)],
                         ssem0)
    for base, nrow in _ROWBLOCKS:
        pltpu.make_async_copy(rows0_v.at[pl.ds(0, nrow)],
                              acc_sh.at[pl.ds(s * ROWS_PER_TILE + base, nrow)],
                              ssem0).wait()
    plsc.subcore_barrier()

    ebase = wid * EDGES_PER_TILE

    def _run_pass(koff, npass):
        st = (
            (src_hbm.at[pl.ds(ebase + koff * CHUNK, npass * CHUNK)],
             src_v.at[pl.ds(0, npass * CHUNK)]),
            (val_hbm.at[pl.ds(ebase + koff * CHUNK, npass * CHUNK)],
             val_v.at[pl.ds(0, npass * CHUNK)]),
            (dst_hbm.at[wid, pl.ds(koff, npass)],
             dst_v.at[pl.ds(0, npass)]),
        )
        for a, bdst in st:
            pltpu.async_copy(a, bdst, gsem0)
        for a, bdst in st:
            pltpu.make_async_copy(a, bdst, gsem0).wait()

        _start_gather(0, 0)
        ntrip = npass // RING

        def _triple(t, carry):
            for b in range(RING):
                k = RING * t + b
                nb = (b + 1) % RING
                if b < 2:
                    @pl.when(t >= 1)
                    def _():
                        _wait_scatter(k - 2, nb)
                    _start_gather(k + 1, nb)
                else:
                    @pl.when(RING * t + RING < npass)
                    def _():
                        _wait_scatter(k - 2, nb)
                        _start_gather(k + 1, nb)
                _wait_gather(k, b)
                _scale(k, b)
                _start_scatter(k, b)
            return carry
        lax.fori_loop(0, ntrip, _triple, 0)

        for k in range(RING * ntrip, npass):
            b = k % RING
            if k + 1 < npass:
                _wait_scatter(k - 2, (b + 1) % RING)
                _start_gather(k + 1, (b + 1) % RING)
            _wait_gather(k, b)
            _scale(k, b)
            _start_scatter(k, b)
        for k in range(npass - RING, npass):
            _wait_scatter(k, k % RING)

    def _pass_loop(p, carry):
        _run_pass(pl.multiple_of(p * PASS, PASS), PASS)
        return carry
    lax.fori_loop(0, NFULL, _pass_loop, 0)
    _run_pass(NFULL * PASS, NCHUNK - NFULL * PASS)

    plsc.subcore_barrier()
    nblk = len(_ROWBLOCKS)

    def _eread(t, b):
        base, nrow = _ROWBLOCKS[t]
        return pltpu.make_async_copy(
            acc_sh.at[pl.ds(s * ROWS_PER_TILE + base, nrow)],
            rows[b].at[pl.ds(0, nrow)], gsem[b])

    def _ewrite(t, b):
        base, nrow = _ROWBLOCKS[t]
        return pltpu.make_async_copy(
            rows[b].at[pl.ds(0, nrow)],
            out_hbm.at[c, pl.ds(s * ROWS_PER_TILE + base, nrow)], ssem[b])

    pltpu.async_copy(acc_sh.at[pl.ds(s * ROWS_PER_TILE, _ROWBLOCKS[0][1])],
                     rows0_v.at[pl.ds(0, _ROWBLOCKS[0][1])], gsem0)
    for t in range(nblk):
        b = t % 2
        _eread(t, b).wait()
        base, nrow = _ROWBLOCKS[t]
        pltpu.async_copy(rows[b].at[pl.ds(0, nrow)],
                         out_hbm.at[c, pl.ds(s * ROWS_PER_TILE + base, nrow)],
                         ssem[b])
        if t + 1 < nblk:
            if t >= 1:
                _ewrite(t - 1, 1 - b).wait()
            nbase, nnrow = _ROWBLOCKS[t + 1]
            pltpu.async_copy(
                acc_sh.at[pl.ds(s * ROWS_PER_TILE + nbase, nnrow)],
                rows[1 - b].at[pl.ds(0, nnrow)], gsem[1 - b])
    _ewrite(nblk - 2, (nblk - 2) % 2).wait()
    _ewrite(nblk - 1, (nblk - 1) % 2).wait()


_spmm_call = functools.partial(
    pl.kernel,
    out_type=jax.ShapeDtypeStruct((NC, NP, D), jnp.float32),
    mesh=plsc.VectorSubcoreMesh(core_axis_name="c", subcore_axis_name="s"),
    scratch_types=[
        pltpu.VMEM((PASS * CHUNK,), jnp.int32),
        pltpu.VMEM((PASS, CHUNK), jnp.int32),
        pltpu.VMEM((PASS * CHUNK,), jnp.float32),
        pltpu.VMEM((CHUNK, D), jnp.float32),
        pltpu.VMEM((CHUNK, D), jnp.float32),
        pltpu.VMEM((CHUNK, D), jnp.float32),
        pltpu.VMEM_SHARED((NP, D), jnp.float32),
        pltpu.SemaphoreType.DMA,
        pltpu.SemaphoreType.DMA,
        pltpu.SemaphoreType.DMA,
        pltpu.SemaphoreType.DMA,
        pltpu.SemaphoreType.DMA,
        pltpu.SemaphoreType.DMA,
    ],
)(_spmm_body)


def _spmm(x, src1d, dst3d, val1d):
    return _spmm_call(x, src1d, dst3d, val1d)


def _bn_body(x_ref, g_ref, b_ref, o_ref):
    x = x_ref[...]
    mu = jnp.mean(x, axis=0, keepdims=True)
    xc = x - mu
    var = jnp.mean(xc * xc, axis=0, keepdims=True)
    o_ref[...] = g_ref[...] * (xc * lax.rsqrt(var + EPS)) + b_ref[...]


def _combine_body(p_ref, o_ref):
    o_ref[...] = p_ref[0, :N, :] + p_ref[1, :N, :]


def _mid_body(xn_ref, y1_ref, y2p_ref, w_ref, b_ref, g_ref, be_ref, o_ref):
    wa = w_ref[0] - w_ref[2]
    wb = w_ref[1]
    wc = 2.0 * w_ref[2]
    o = (jnp.dot(xn_ref[...], wa, preferred_element_type=jnp.float32)
         + jnp.dot(y1_ref[...], wb, preferred_element_type=jnp.float32)
         + jnp.dot(y2p_ref[0, :N, :] + y2p_ref[1, :N, :], wc,
                   preferred_element_type=jnp.float32)
         + b_ref[...])
    o = jnp.maximum(o, 0.0)
    mu = jnp.mean(o, axis=0, keepdims=True)
    oc = o - mu
    var = jnp.mean(oc * oc, axis=0, keepdims=True)
    o_ref[...] = g_ref[...] * (oc * lax.rsqrt(var + EPS)) + be_ref[...]


def _final_body(z_ref, y3_ref, y4p_ref, w_ref, b_ref, xn_ref, o_ref):
    wa = w_ref[0] - w_ref[2]
    wb = w_ref[1]
    wc = 2.0 * w_ref[2]
    o = (jnp.dot(z_ref[...], wa, preferred_element_type=jnp.float32)
         + jnp.dot(y3_ref[...], wb, preferred_element_type=jnp.float32)
         + jnp.dot(y4p_ref[0, :N, :] + y4p_ref[1, :N, :], wc,
                   preferred_element_type=jnp.float32)
         + b_ref[...])
    o_ref[...] = jnp.maximum(o + xn_ref[...], 0.0)


_ND = jax.ShapeDtypeStruct((N, D), jnp.float32)


def kernel(x, lap_indices, lap_values, g1, be1, W1, b1, g2, be2, W2, b2):
    src1d = lap_indices[1]
    dst3d = lap_indices[0].reshape(NW, NCHUNK, CHUNK)
    val1d = lap_values
    g1r, be1r, b1r = g1.reshape(1, D), be1.reshape(1, D), b1.reshape(1, D)
    g2r, be2r, b2r = g2.reshape(1, D), be2.reshape(1, D), b2.reshape(1, D)

    xn = pl.pallas_call(_bn_body, out_shape=_ND)(x, g1r, be1r)
    y1p = _spmm(xn, src1d, dst3d, val1d)
    y1 = pl.pallas_call(_combine_body, out_shape=_ND)(y1p)
    y2p = _spmm(y1, src1d, dst3d, val1d)
    z = pl.pallas_call(_mid_body, out_shape=_ND)(
        xn, y1, y2p, W1, b1r, g2r, be2r)
    y3p = _spmm(z, src1d, dst3d, val1d)
    y3 = pl.pallas_call(_combine_body, out_shape=_ND)(y3p)
    y4p = _spmm(y3, src1d, dst3d, val1d)
    out = pl.pallas_call(_final_body, out_shape=_ND)(
        z, y3, y4p, W2, b2r, xn)
    return out


# speedup vs baseline: 1.7183x; 1.0148x over previous
import functools

import jax
import jax.numpy as jnp
from jax import lax
from jax.experimental import pallas as pl
from jax.experimental.pallas import tpu as pltpu
from jax.experimental.pallas import tpu_sc as plsc

N = 10000
E = 320000
D = 128
EPS = 1e-5

NC = 2
NS = 16
NW = NC * NS
EDGES_RAW = E // NW
CHUNK = 80
NCHUNK = 125
EDGES_PER_TILE = NCHUNK * CHUNK
PASS = 64
RING = 3
NP = 10112
ROWS_PER_TILE = NP // NS
LANES = 16
DSTEP = D // LANES
_ROWBLOCKS = [(t * CHUNK, CHUNK) for t in range(ROWS_PER_TILE // CHUNK)]
if ROWS_PER_TILE % CHUNK:
    _ROWBLOCKS.append((ROWS_PER_TILE // CHUNK * CHUNK,
                       ROWS_PER_TILE % CHUNK))


def _spmm_body(x_hbm, src_hbm, dst_hbm, val_hbm, out_hbm,
               src_v, dst_v, val_v, rows0_v, rows1_v, rows2_v,
               acc_sh, gsem0, gsem1, gsem2, ssem0, ssem1, ssem2):
    c = lax.axis_index("c")
    s = lax.axis_index("s")
    wid = c * NS + s
    rows = (rows0_v, rows1_v, rows2_v)
    gsem = (gsem0, gsem1, gsem2)
    ssem = (ssem0, ssem1, ssem2)

    def _start_gather(k, b):
        pltpu.async_copy(x_hbm.at[src_v.at[pl.ds(k * CHUNK, CHUNK)]],
                         rows[b], gsem[b])

    def _wait_gather(k, b):
        pltpu.make_async_copy(x_hbm.at[src_v.at[pl.ds(k * CHUNK, CHUNK)]],
                              rows[b], gsem[b]).wait()

    def _start_scatter(k, b):
        pltpu.async_copy(rows[b], acc_sh.at[dst_v.at[k]], ssem[b], add=True)

    def _wait_scatter(k, b):
        pltpu.make_async_copy(rows[b], acc_sh.at[dst_v.at[k]], ssem[b]).wait()

    def _scale(k, b):
        rb = rows[b]
        def _group(g, carry):
            vv = val_v[pl.ds(k * CHUNK + g * LANES, LANES)]
            for e in range(LANES):
                v = vv[e]
                i = g * LANES + e
                for j in range(DSTEP):
                    sl = pl.ds(j * LANES, LANES)
                    rb[i, sl] = rb[i, sl] * v
            return carry
        lax.fori_loop(0, CHUNK // LANES, _group, 0)

    def _zrow(r, carry):
        for j in range(DSTEP):
            rows0_v[r, pl.ds(j * LANES, LANES)] = jnp.zeros((LANES,),
                                                            jnp.float32)
        return carry
    lax.fori_loop(0, CHUNK, _zrow, 0)
    for base, nrow in _ROWBLOCKS:
        pltpu.async_copy(rows0_v.at[pl.ds(0, nrow)],
                         acc_sh.at[pl.ds(s * ROWS_PER_TILE + base, nrow)],
                         ssem0)
    for base, nrow in _ROWBLOCKS:
        pltpu.make_async_copy(rows0_v.at[pl.ds(0, nrow)],
                              acc_sh.at[pl.ds(s * ROWS_PER_TILE + base, nrow)],
                              ssem0).wait()
    plsc.subcore_barrier()

    ebase = wid * EDGES_PER_TILE

    def _run_pass(koff, npass):
        st = (
            (src_hbm.at[pl.ds(ebase + koff * CHUNK, npass * CHUNK)],
             src_v.at[pl.ds(0, npass * CHUNK)]),
            (val_hbm.at[pl.ds(ebase + koff * CHUNK, npass * CHUNK)],
             val_v.at[pl.ds(0, npass * CHUNK)]),
            (dst_hbm.at[wid, pl.ds(koff, npass)],
             dst_v.at[pl.ds(0, npass)]),
        )
        for a, bdst in st:
            pltpu.async_copy(a, bdst, gsem0)
        for a, bdst in st:
            pltpu.make_async_copy(a, bdst, gsem0).wait()

        _start_gather(0, 0)
        ntrip = npass // RING

        def _triple(t, carry):
            for b in range(RING):
                k = RING * t + b
                nb = (b + 1) % RING
                if b < 2:
                    @pl.when(t >= 1)
                    def _():
                        _wait_scatter(k - 2, nb)
                    _start_gather(k + 1, nb)
                else:
                    @pl.when(RING * t + RING < npass)
                    def _():
                        _wait_scatter(k - 2, nb)
                        _start_gather(k + 1, nb)
                _wait_gather(k, b)
                _scale(k, b)
                _start_scatter(k, b)
            return carry
        lax.fori_loop(0, ntrip, _triple, 0)

        for k in range(RING * ntrip, npass):
            b = k % RING
            if k + 1 < npass:
                _wait_scatter(k - 2, (b + 1) % RING)
                _start_gather(k + 1, (b + 1) % RING)
            _wait_gather(k, b)
            _scale(k, b)
            _start_scatter(k, b)
        for k in range(npass - RING, npass):
            _wait_scatter(k, k % RING)

    _run_pass(0, PASS)
    _run_pass(PASS, NCHUNK - PASS)

    plsc.subcore_barrier()
    nblk = len(_ROWBLOCKS)

    def _eread(t, b):
        base, nrow = _ROWBLOCKS[t]
        return pltpu.make_async_copy(
            acc_sh.at[pl.ds(s * ROWS_PER_TILE + base, nrow)],
            rows[b].at[pl.ds(0, nrow)], gsem[b])

    def _ewrite(t, b):
        base, nrow = _ROWBLOCKS[t]
        return pltpu.make_async_copy(
            rows[b].at[pl.ds(0, nrow)],
            out_hbm.at[c, pl.ds(s * ROWS_PER_TILE + base, nrow)], ssem[b])

    pltpu.async_copy(acc_sh.at[pl.ds(s * ROWS_PER_TILE, _ROWBLOCKS[0][1])],
                     rows0_v.at[pl.ds(0, _ROWBLOCKS[0][1])], gsem0)
    for t in range(nblk):
        b = t % 2
        _eread(t, b).wait()
        base, nrow = _ROWBLOCKS[t]
        pltpu.async_copy(rows[b].at[pl.ds(0, nrow)],
                         out_hbm.at[c, pl.ds(s * ROWS_PER_TILE + base, nrow)],
                         ssem[b])
        if t + 1 < nblk:
            if t >= 1:
                _ewrite(t - 1, 1 - b).wait()
            nbase, nnrow = _ROWBLOCKS[t + 1]
            pltpu.async_copy(
                acc_sh.at[pl.ds(s * ROWS_PER_TILE + nbase, nnrow)],
                rows[1 - b].at[pl.ds(0, nnrow)], gsem[1 - b])
    _ewrite(nblk - 2, (nblk - 2) % 2).wait()
    _ewrite(nblk - 1, (nblk - 1) % 2).wait()


_spmm_call = functools.partial(
    pl.kernel,
    out_type=jax.ShapeDtypeStruct((NC, NP, D), jnp.float32),
    mesh=plsc.VectorSubcoreMesh(core_axis_name="c", subcore_axis_name="s"),
    scratch_types=[
        pltpu.VMEM((PASS * CHUNK,), jnp.int32),
        pltpu.VMEM((PASS, CHUNK), jnp.int32),
        pltpu.VMEM((PASS * CHUNK,), jnp.float32),
        pltpu.VMEM((CHUNK, D), jnp.float32),
        pltpu.VMEM((CHUNK, D), jnp.float32),
        pltpu.VMEM((CHUNK, D), jnp.float32),
        pltpu.VMEM_SHARED((NP, D), jnp.float32),
        pltpu.SemaphoreType.DMA,
        pltpu.SemaphoreType.DMA,
        pltpu.SemaphoreType.DMA,
        pltpu.SemaphoreType.DMA,
        pltpu.SemaphoreType.DMA,
        pltpu.SemaphoreType.DMA,
    ],
)(_spmm_body)


def _spmm(x, src1d, dst3d, val1d):
    return _spmm_call(x, src1d, dst3d, val1d)


def _bn_body(x_ref, g_ref, b_ref, o_ref):
    x = x_ref[...]
    mu = jnp.mean(x, axis=0, keepdims=True)
    xc = x - mu
    var = jnp.mean(xc * xc, axis=0, keepdims=True)
    o_ref[...] = g_ref[...] * (xc * lax.rsqrt(var + EPS)) + b_ref[...]


def _combine_body(p_ref, o_ref):
    o_ref[...] = p_ref[0, :N, :] + p_ref[1, :N, :]


def _mid_body(xn_ref, y1_ref, y2p_ref, w_ref, b_ref, g_ref, be_ref, o_ref):
    wa = w_ref[0] - w_ref[2]
    wb = w_ref[1]
    wc = 2.0 * w_ref[2]
    o = (jnp.dot(xn_ref[...], wa, preferred_element_type=jnp.float32)
         + jnp.dot(y1_ref[...], wb, preferred_element_type=jnp.float32)
         + jnp.dot(y2p_ref[0, :N, :] + y2p_ref[1, :N, :], wc,
                   preferred_element_type=jnp.float32)
         + b_ref[...])
    o = jnp.maximum(o, 0.0)
    mu = jnp.mean(o, axis=0, keepdims=True)
    oc = o - mu
    var = jnp.mean(oc * oc, axis=0, keepdims=True)
    o_ref[...] = g_ref[...] * (oc * lax.rsqrt(var + EPS)) + be_ref[...]


def _final_body(z_ref, y3_ref, y4p_ref, w_ref, b_ref, xn_ref, o_ref):
    wa = w_ref[0] - w_ref[2]
    wb = w_ref[1]
    wc = 2.0 * w_ref[2]
    o = (jnp.dot(z_ref[...], wa, preferred_element_type=jnp.float32)
         + jnp.dot(y3_ref[...], wb, preferred_element_type=jnp.float32)
         + jnp.dot(y4p_ref[0, :N, :] + y4p_ref[1, :N, :], wc,
                   preferred_element_type=jnp.float32)
         + b_ref[...])
    o_ref[...] = jnp.maximum(o + xn_ref[...], 0.0)


_ND = jax.ShapeDtypeStruct((N, D), jnp.float32)


def kernel(x, lap_indices, lap_values, g1, be1, W1, b1, g2, be2, W2, b2):
    src1d = lap_indices[1]
    dst3d = lap_indices[0].reshape(NW, NCHUNK, CHUNK)
    val1d = lap_values
    g1r, be1r, b1r = g1.reshape(1, D), be1.reshape(1, D), b1.reshape(1, D)
    g2r, be2r, b2r = g2.reshape(1, D), be2.reshape(1, D), b2.reshape(1, D)

    xn = pl.pallas_call(_bn_body, out_shape=_ND)(x, g1r, be1r)
    y1p = _spmm(xn, src1d, dst3d, val1d)
    y1 = pl.pallas_call(_combine_body, out_shape=_ND)(y1p)
    y2p = _spmm(y1, src1d, dst3d, val1d)
    z = pl.pallas_call(_mid_body, out_shape=_ND)(
        xn, y1, y2p, W1, b1r, g2r, be2r)
    y3p = _spmm(z, src1d, dst3d, val1d)
    y3 = pl.pallas_call(_combine_body, out_shape=_ND)(y3p)
    y4p = _spmm(y3, src1d, dst3d, val1d)
    out = pl.pallas_call(_final_body, out_shape=_ND)(
        z, y3, y4p, W2, b2r, xn)
    return out
